# Initial kernel scaffold; baseline (speedup 1.0000x reference)
import jax
import jax.numpy as jnp
from jax.experimental import pallas as pl


def kernel(x, plane):
    raise NotImplementedError("write your pallas kernel here")


# speedup vs baseline: 35.5370x; 35.5370x over previous
import functools

import jax
import jax.numpy as jnp
from jax import lax
from jax.experimental import pallas as pl
from jax.experimental.pallas import tpu as pltpu
from jax.experimental.pallas import tpu_sc as plsc

_NC = 2
_NS = 16
_L = 16
_NW = _NC * _NS


def _make_sc_kernel(P, H, W, C, N):
    PW = P // _NW
    n_chunks = PW // N
    n_groups = N // _L

    mesh = plsc.VectorSubcoreMesh(
        core_axis_name="c", subcore_axis_name="s",
        num_cores=_NC, num_subcores=_NS)

    def body(xs_hbm, ys_hbm, table_hbm, out_hbm,
             xs_v, ys_v,
             i00_v, i01_v, i10_v, i11_v,
             w00_v, w01_v, w10_v, w11_v,
             r00_v, r01_v, r10_v, r11_v,
             out_v, sem):
        wid = lax.axis_index("s") * _NC + lax.axis_index("c")

        wf = jnp.float32(W)
        hf = jnp.float32(H)

        def axis_terms(v, extent):
            ip = ((v + 1.0) * extent - 1.0) * 0.5
            t0 = ip.astype(jnp.int32)
            t0 = jnp.where(t0.astype(jnp.float32) > ip, t0 - 1, t0)
            f0 = t0.astype(jnp.float32)
            w1 = ip - f0
            w0 = 1.0 - w1
            t1 = t0 + 1
            lim = extent - 1.0
            in0 = (f0 >= 0.0) & (f0 <= lim)
            in1 = (f0 + 1.0 >= 0.0) & (f0 + 1.0 <= lim)
            w0 = jnp.where(in0, w0, 0.0)
            w1 = jnp.where(in1, w1, 0.0)
            ilim = jnp.int32(extent) - 1
            c0 = jnp.minimum(jnp.maximum(t0, 0), ilim)
            c1 = jnp.minimum(jnp.maximum(t1, 0), ilim)
            return c0, c1, w0, w1

        def chunk_body(t, carry):
            base = wid * PW + t * N
            pltpu.sync_copy(xs_hbm.at[pl.ds(base, N)], xs_v)
            pltpu.sync_copy(ys_hbm.at[pl.ds(base, N)], ys_v)

            def group_a(g, c2):
                off = g * _L
                xv = xs_v[pl.ds(off, _L)]
                yv = ys_v[pl.ds(off, _L)]
                x0, x1, wx0, wx1 = axis_terms(xv, wf)
                y0, y1, wy0, wy1 = axis_terms(yv, hf)
                r0 = y0 * W
                r1 = y1 * W
                i00_v[pl.ds(off, _L)] = r0 + x0
                i01_v[pl.ds(off, _L)] = r0 + x1
                i10_v[pl.ds(off, _L)] = r1 + x0
                i11_v[pl.ds(off, _L)] = r1 + x1
                w00_v[pl.ds(off, _L)] = wx0 * wy0
                w01_v[pl.ds(off, _L)] = wx1 * wy0
                w10_v[pl.ds(off, _L)] = wx0 * wy1
                w11_v[pl.ds(off, _L)] = wx1 * wy1
                return c2

            lax.fori_loop(0, n_groups, group_a, 0, unroll=True)

            c0 = pltpu.async_copy(table_hbm.at[i00_v], r00_v, sem)
            c1 = pltpu.async_copy(table_hbm.at[i01_v], r01_v, sem)
            c2 = pltpu.async_copy(table_hbm.at[i10_v], r10_v, sem)
            c3 = pltpu.async_copy(table_hbm.at[i11_v], r11_v, sem)
            c0.wait()
            c1.wait()
            c2.wait()
            c3.wait()

            for p in range(N):
                g = p // _L
                j = p % _L
                if j == 0:
                    w00g = w00_v[pl.ds(g * _L, _L)]
                    w01g = w01_v[pl.ds(g * _L, _L)]
                    w10g = w10_v[pl.ds(g * _L, _L)]
                    w11g = w11_v[pl.ds(g * _L, _L)]
                w00 = w00g[j]
                w01 = w01g[j]
                w10 = w10g[j]
                w11 = w11g[j]
                for h in range(0, C, _L):
                    s = pl.ds(h, _L)
                    val = (w00 * r00_v[p, s] + w01 * r01_v[p, s]
                           + w10 * r10_v[p, s] + w11 * r11_v[p, s])
                    out_v[p, s] = val

            pltpu.sync_copy(out_v, out_hbm.at[pl.ds(base, N)])
            return carry

        lax.fori_loop(0, n_chunks, chunk_body, 0)

    return pl.kernel(
        body,
        out_type=jax.ShapeDtypeStruct((P, C), jnp.float32),
        mesh=mesh,
        compiler_params=pltpu.CompilerParams(use_tc_tiling_on_sc=False),
        scratch_types=[
            pltpu.VMEM((N,), jnp.float32),
            pltpu.VMEM((N,), jnp.float32),
            pltpu.VMEM((N,), jnp.int32),
            pltpu.VMEM((N,), jnp.int32),
            pltpu.VMEM((N,), jnp.int32),
            pltpu.VMEM((N,), jnp.int32),
            pltpu.VMEM((N,), jnp.float32),
            pltpu.VMEM((N,), jnp.float32),
            pltpu.VMEM((N,), jnp.float32),
            pltpu.VMEM((N,), jnp.float32),
            pltpu.VMEM((N, C), jnp.float32),
            pltpu.VMEM((N, C), jnp.float32),
            pltpu.VMEM((N, C), jnp.float32),
            pltpu.VMEM((N, C), jnp.float32),
            pltpu.VMEM((N, C), jnp.float32),
            pltpu.SemaphoreType.DMA,
        ],
    )


@functools.partial(jax.jit, static_argnames=())
def kernel(x, plane):
    C = plane.shape[1]
    H = plane.shape[2]
    W = plane.shape[3]
    pts = x.reshape(-1, 2)
    P = pts.shape[0]
    table = jnp.transpose(plane.reshape(C, H * W))
    xs = pts[:, 0]
    ys = pts[:, 1]
    sc = _make_sc_kernel(P, H, W, C, 128)
    out = sc(xs, ys, table)
    return out.reshape(x.shape[:-1] + (C,))


# speedup vs baseline: 68.3417x; 1.9231x over previous
import functools

import jax
import jax.numpy as jnp
from jax import lax
from jax.experimental import pallas as pl
from jax.experimental.pallas import tpu as pltpu
from jax.experimental.pallas import tpu_sc as plsc

_NC = 2
_NS = 16
_L = 16
_NW = _NC * _NS


def _make_sc_kernel(P, H, W, C, N):
    PW = P // _NW
    n_chunks = PW // N
    n_groups = N // _L
    assert n_chunks % 2 == 0

    mesh = plsc.VectorSubcoreMesh(
        core_axis_name="c", subcore_axis_name="s",
        num_cores=_NC, num_subcores=_NS)

    wf = jnp.float32(W)
    hf = jnp.float32(H)

    def axis_terms(v, extent):
        ip = ((v + 1.0) * extent - 1.0) * 0.5
        t0 = ip.astype(jnp.int32)
        t0 = jnp.where(t0.astype(jnp.float32) > ip, t0 - 1, t0)
        f0 = t0.astype(jnp.float32)
        w1 = ip - f0
        w0 = 1.0 - w1
        t1 = t0 + 1
        lim = extent - 1.0
        in0 = (f0 >= 0.0) & (f0 <= lim)
        in1 = (f0 + 1.0 >= 0.0) & (f0 + 1.0 <= lim)
        w0 = jnp.where(in0, w0, 0.0)
        w1 = jnp.where(in1, w1, 0.0)
        ilim = jnp.int32(extent) - 1
        c0 = jnp.minimum(jnp.maximum(t0, 0), ilim)
        c1 = jnp.minimum(jnp.maximum(t1, 0), ilim)
        return c0, c1, w0, w1

    def body(xs_hbm, ys_hbm, table_hbm, out_hbm, *refs):
        it = iter(refs)
        xs_v = [next(it) for _ in range(2)]
        ys_v = [next(it) for _ in range(2)]
        idx_v = [[next(it) for _ in range(4)] for _ in range(2)]
        w_v = [[next(it) for _ in range(4)] for _ in range(2)]
        rows_v = [[next(it) for _ in range(4)] for _ in range(2)]
        out_v = [next(it) for _ in range(2)]
        psem = [next(it) for _ in range(2)]
        gsem = [next(it) for _ in range(2)]
        osem = [next(it) for _ in range(2)]

        wid = lax.axis_index("s") * _NC + lax.axis_index("c")
        base0 = wid * PW

        def pts_fire(t, s):
            pltpu.async_copy(xs_hbm.at[pl.ds(base0 + t * N, N)], xs_v[s], psem[s])
            pltpu.async_copy(ys_hbm.at[pl.ds(base0 + t * N, N)], ys_v[s], psem[s])

        def pts_wait(s):
            pltpu.make_async_copy(xs_hbm.at[pl.ds(0, N)], xs_v[s], psem[s]).wait()
            pltpu.make_async_copy(ys_hbm.at[pl.ds(0, N)], ys_v[s], psem[s]).wait()

        def compute_idx(s):
            def group_a(g, carry):
                off = g * _L
                xv = xs_v[s][pl.ds(off, _L)]
                yv = ys_v[s][pl.ds(off, _L)]
                x0, x1, wx0, wx1 = axis_terms(xv, wf)
                y0, y1, wy0, wy1 = axis_terms(yv, hf)
                r0 = y0 * W
                r1 = y1 * W
                idx_v[s][0][pl.ds(off, _L)] = r0 + x0
                idx_v[s][1][pl.ds(off, _L)] = r0 + x1
                idx_v[s][2][pl.ds(off, _L)] = r1 + x0
                idx_v[s][3][pl.ds(off, _L)] = r1 + x1
                w_v[s][0][pl.ds(off, _L)] = wx0 * wy0
                w_v[s][1][pl.ds(off, _L)] = wx1 * wy0
                w_v[s][2][pl.ds(off, _L)] = wx0 * wy1
                w_v[s][3][pl.ds(off, _L)] = wx1 * wy1
                return carry
            lax.fori_loop(0, n_groups, group_a, 0)

        def gathers_fire(s):
            for k in range(4):
                pltpu.async_copy(table_hbm.at[idx_v[s][k]], rows_v[s][k], gsem[s])

        def gathers_wait(s):
            for k in range(4):
                pltpu.make_async_copy(
                    table_hbm.at[idx_v[s][k]], rows_v[s][k], gsem[s]).wait()

        def blend(s):
            def group_b(g, carry):
                off = g * _L
                w00g = w_v[s][0][pl.ds(off, _L)]
                w01g = w_v[s][1][pl.ds(off, _L)]
                w10g = w_v[s][2][pl.ds(off, _L)]
                w11g = w_v[s][3][pl.ds(off, _L)]
                for j in range(_L):
                    p = off + j
                    w00 = w00g[j]
                    w01 = w01g[j]
                    w10 = w10g[j]
                    w11 = w11g[j]
                    for h in range(0, C, _L):
                        sl = pl.ds(h, _L)
                        val = (w00 * rows_v[s][0][p, sl]
                               + w01 * rows_v[s][1][p, sl]
                               + w10 * rows_v[s][2][p, sl]
                               + w11 * rows_v[s][3][p, sl])
                        out_v[s][p, sl] = val
                return carry
            lax.fori_loop(0, n_groups, group_b, 0)

        def out_fire(t, s):
            pltpu.async_copy(out_v[s], out_hbm.at[pl.ds(base0 + t * N, N)], osem[s])

        def out_wait(s):
            pltpu.make_async_copy(
                out_v[s], out_hbm.at[pl.ds(0, N)], osem[s]).wait()

        pts_fire(0, 0)
        pts_fire(1, 1)
        pts_wait(0)
        compute_idx(0)
        gathers_fire(0)

        def pair_body(q, carry):
            for par in (0, 1):
                t = 2 * q + par
                nxt = 1 - par

                @pl.when(t + 2 < n_chunks)
                def _():
                    pts_fire(t + 2, par)

                @pl.when(t + 1 < n_chunks)
                def _():
                    pts_wait(nxt)
                    compute_idx(nxt)
                    gathers_fire(nxt)

                gathers_wait(par)

                @pl.when(t >= 2)
                def _():
                    out_wait(par)

                blend(par)
                out_fire(t, par)
            return carry

        lax.fori_loop(0, n_chunks // 2, pair_body, 0)

        out_wait(0)
        out_wait(1)

    scratch = (
        [pltpu.VMEM((N,), jnp.float32) for _ in range(2)]
        + [pltpu.VMEM((N,), jnp.float32) for _ in range(2)]
        + [pltpu.VMEM((N,), jnp.int32) for _ in range(8)]
        + [pltpu.VMEM((N,), jnp.float32) for _ in range(8)]
        + [pltpu.VMEM((N, C), jnp.float32) for _ in range(8)]
        + [pltpu.VMEM((N, C), jnp.float32) for _ in range(2)]
        + [pltpu.SemaphoreType.DMA for _ in range(6)]
    )

    return pl.kernel(
        body,
        out_type=jax.ShapeDtypeStruct((P, C), jnp.float32),
        mesh=mesh,
        compiler_params=pltpu.CompilerParams(use_tc_tiling_on_sc=False),
        scratch_types=scratch,
    )


@functools.partial(jax.jit, static_argnames=())
def kernel(x, plane):
    C = plane.shape[1]
    H = plane.shape[2]
    W = plane.shape[3]
    pts = x.reshape(-1, 2)
    P = pts.shape[0]
    table = jnp.transpose(plane.reshape(C, H * W))
    xs = pts[:, 0]
    ys = pts[:, 1]
    sc = _make_sc_kernel(P, H, W, C, 128)
    out = sc(xs, ys, table)
    return out.reshape(x.shape[:-1] + (C,))
